# baseline (speedup 1.0000x reference)
import functools

import jax
import jax.numpy as jnp
from jax import lax
from jax.experimental import pallas as pl
from jax.experimental.pallas import tpu as pltpu
from jax.experimental.pallas import tpu_sc as plsc

D_MODEL = 2048
SEQ_LEN = 2048

NUM_CORES = 2
NUM_SUBCORES = 16
NUM_WORKERS = NUM_CORES * NUM_SUBCORES

CHUNK = 16


def _sc_embed(seq_flat, table, pe):
    num_tokens = seq_flat.shape[0]
    per_worker = num_tokens // NUM_WORKERS
    num_chunks = per_worker // CHUNK
    mesh = plsc.VectorSubcoreMesh(core_axis_name="c", subcore_axis_name="s")

    @functools.partial(
        pl.kernel,
        out_type=jax.ShapeDtypeStruct((num_tokens, D_MODEL), jnp.float32),
        mesh=mesh,
        scratch_types=[
            pltpu.VMEM((per_worker,), jnp.int32),
            pltpu.VMEM((CHUNK, D_MODEL), jnp.float32),
            pltpu.VMEM((CHUNK, D_MODEL), jnp.float32),
            pltpu.SemaphoreType.DMA,
        ],
    )
    def k(seq_hbm, table_hbm, pe_hbm, out_hbm, idx_v, rows_v, pe_v, sem):
        wid = lax.axis_index("s") * NUM_CORES + lax.axis_index("c")
        base = wid * per_worker
        pos0 = lax.rem(base, SEQ_LEN)
        pltpu.sync_copy(seq_hbm.at[pl.ds(base, per_worker)], idx_v)
        for c in range(num_chunks):
            gather = pltpu.async_copy(
                table_hbm.at[idx_v.at[pl.ds(c * CHUNK, CHUNK)]], rows_v, sem
            )
            pltpu.sync_copy(pe_hbm.at[pl.ds(pos0 + c * CHUNK, CHUNK)], pe_v)
            gather.wait()

            def add_row(j, carry):
                for i in range(D_MODEL // 16):
                    plsc.addupdate(
                        rows_v.at[j, pl.ds(i * 16, 16)],
                        pe_v[j, pl.ds(i * 16, 16)],
                    )
                return carry

            lax.fori_loop(0, CHUNK, add_row, 0)
            pltpu.sync_copy(rows_v, out_hbm.at[pl.ds(base + c * CHUNK, CHUNK)])

    return k(seq_flat, table, pe)


def kernel(seq, table, pe):
    batch, seq_len = seq.shape
    seq_flat = seq.reshape(-1).astype(jnp.int32)
    out = _sc_embed(seq_flat, table, pe)
    return out.reshape(batch, seq_len, D_MODEL)
